# Initial kernel scaffold; baseline (speedup 1.0000x reference)
import jax
import jax.numpy as jnp
from jax.experimental import pallas as pl


def kernel(h, lice, node_features, edge_index, edge_distance, edge_direction, We1, be1, We2, be2, Wa1, ba1, Wa2, ba2, log_beta):
    raise NotImplementedError("write your pallas kernel here")


# speedup vs baseline: 8.7023x; 8.7023x over previous
import functools

import jax
import jax.numpy as jnp
from jax import lax
from jax.experimental import pallas as pl
from jax.experimental.pallas import tpu as pltpu
from jax.experimental.pallas import tpu_sc as plsc

N = 50000
E = 800000
H = 64
DECAY_KM = 15.0
NEG = -3.0e38

NC = 2
NS = 16
NW = NC * NS

EP_TILE = 25088
EPAD = EP_TILE * NW
ECHUNK = 512
NCHUNK = EP_TILE // ECHUNK
GCHUNK = 128

NPAD = 51200
NSLICE = NPAD // NS
NGRP = NSLICE // 16

BE = 2048
NB = EPAD // BE

HALF = NPAD // NC
ROWS_TILE = HALF // NS
PB_ROWS = HALF + 128

_MESH = plsc.VectorSubcoreMesh(core_axis_name="c", subcore_axis_name="s")
_SC_PARAMS = pltpu.CompilerParams(needs_layout_passes=False)
_SC_PARAMS_UNTILED = pltpu.CompilerParams(
    needs_layout_passes=False, use_tc_tiling_on_sc=False)

_f32 = jnp.float32
_i32 = jnp.int32


def _wid():
    return lax.axis_index("s") * NC + lax.axis_index("c")


def _lane():
    return lax.iota(_i32, 16)


def _fill(ref, n, value, dtype):
    v = jnp.full((16,), value, dtype)

    def body(i, _):
        ref[pl.ds(i * 16, 16)] = v
        return 0

    lax.fori_loop(0, n // 16, body, 0)


def _seg_rmw(arr_ref, kb_ref, vb_ref, k, v, op):
    ks, vs = plsc.sort_key_val(k, v)
    lane = _lane()
    kb_ref[...] = ks
    for off in (1, 2, 4, 8):
        idx = jnp.maximum(lane - off, 0)
        vb_ref[...] = vs
        ksh = plsc.load_gather(kb_ref, [idx])
        vsh = plsc.load_gather(vb_ref, [idx])
        vs = jnp.where((ksh == ks) & (lane >= off), op(vs, vsh), vs)
    knext = plsc.load_gather(kb_ref, [jnp.minimum(lane + 1, 15)])
    is_last = (knext != ks) | (lane == 15)
    old = plsc.load_gather(arr_ref, [ks])
    plsc.store_scatter(arr_ref, [ks], op(old, vs), mask=is_last)


def _merge_to_out(arr_ref, sp_ref, row_ref, acc_ref, out_ref, cid, sid, op):
    stage = cid * NS * NPAD
    pltpu.sync_copy(arr_ref, sp_ref.at[pl.ds(stage + sid * NPAD, NPAD)])
    plsc.subcore_barrier()
    base = sid * NSLICE
    pltpu.sync_copy(sp_ref.at[pl.ds(stage + base, NSLICE)], acc_ref)
    for r in range(1, NS):
        pltpu.sync_copy(sp_ref.at[pl.ds(stage + r * NPAD + base, NSLICE)],
                        row_ref)

        def body(j, _):
            s = pl.ds(j * 16, 16)
            acc_ref[s] = op(acc_ref[s], row_ref[s])
            return 0

        lax.fori_loop(0, NGRP, body, 0)
    pltpu.sync_copy(acc_ref, out_ref.at[pl.ds(cid * NPAD + base, NSLICE)])


def _merge_parts(part_hbm, full_ref, row_ref, op):
    pltpu.sync_copy(part_hbm.at[pl.ds(0, NPAD)], full_ref)
    for r in range(NS):
        pltpu.sync_copy(part_hbm.at[pl.ds(NPAD + r * NSLICE, NSLICE)], row_ref)

        def body(j, _):
            s = pl.ds(r * NSLICE + j * 16, 16)
            t = pl.ds(j * 16, 16)
            full_ref[s] = op(full_ref[s], row_ref[t])
            return 0

        lax.fori_loop(0, NGRP, body, 0)


@functools.partial(
    pl.kernel,
    out_type=jax.ShapeDtypeStruct((EPAD, 128), _f32),
    mesh=_MESH,
    compiler_params=_SC_PARAMS,
    scratch_types=[
        pltpu.VMEM((GCHUNK,), _i32),
        pltpu.VMEM((GCHUNK, 128), _f32),
        pltpu.SemaphoreType.DMA,
    ],
)
def _sc_gather(src_hbm, htab_hbm, hx_hbm, idx_v, rows_v, sem):
    base = _wid() * EP_TILE

    def body(i, _):
        off = base + i * GCHUNK
        pltpu.sync_copy(src_hbm.at[pl.ds(off, GCHUNK)], idx_v)
        pltpu.async_copy(htab_hbm.at[idx_v], rows_v, sem).wait()
        pltpu.sync_copy(rows_v, hx_hbm.at[pl.ds(off, GCHUNK)])
        return 0

    lax.fori_loop(0, EP_TILE // GCHUNK, body, 0)


def _tc_logits_body(dist, d0, d1, hx, we1, be1, we2, be2,
                    wa1e, wa1h, ba1, wa2, ba2, out, scale_out):
    temp = hx[:, 64:65]
    sal = hx[:, 65:66]
    u = hx[:, 66:67]
    v = hx[:, 67:68]
    hsrc = hx[:, 0:H]
    flux = u * d0[...] + v * d1[...]
    wf = jnp.maximum(flux, 0.0) * jnp.exp(-dist[...] / DECAY_KM)
    e1 = jnp.maximum(
        dist[...] * we1[0:1, :] + wf * we1[1:2, :]
        + temp * we1[2:3, :] + sal * we1[3:4, :] + be1[...], 0.0)
    w_ea = jnp.dot(we2[...], wa1e[...], preferred_element_type=_f32)
    b_fold = jnp.dot(be2[...], wa1e[...], preferred_element_type=_f32) + ba1[...]
    att = jnp.maximum(
        jnp.dot(e1, w_ea, preferred_element_type=_f32)
        + jnp.dot(hsrc, wa1h[...], preferred_element_type=_f32)
        + b_fold, 0.0)
    logit = (jnp.sum(att * wa2[...], axis=1, keepdims=True) + ba2[...]
             + jnp.log(wf + 1e-8))
    rid = pl.program_id(0) * BE + lax.broadcasted_iota(_i32, (BE, 1), 0)
    out[...] = jnp.where(rid < E, logit, NEG)
    scale_out[...] = hx[:, 68:69]


def _tc_logits(dist, d0, d1, hx, We1, be1, We2, be2, Wa1, ba1, Wa2, ba2):
    col = pl.BlockSpec((BE, 1), lambda i: (i, 0))
    full = lambda a: pl.BlockSpec(a.shape, lambda i: tuple(0 for _ in a.shape))
    args = (dist[:, None], d0[:, None], d1[:, None], hx,
            We1, be1[None, :], We2, be2[None, :],
            Wa1[:H], Wa1[H:], ba1[None, :], Wa2[:, 0][None, :],
            ba2[None, :])
    logits, scale = pl.pallas_call(
        _tc_logits_body,
        grid=(NB,),
        in_specs=[col, col, col,
                  pl.BlockSpec((BE, 128), lambda i: (i, 0)),
                  full(args[4]), full(args[5]), full(args[6]), full(args[7]),
                  full(args[8]), full(args[9]), full(args[10]),
                  full(args[11]), full(args[12])],
        out_specs=(col, col),
        out_shape=(jax.ShapeDtypeStruct((EPAD, 1), _f32),
                   jax.ShapeDtypeStruct((EPAD, 1), _f32)),
    )(*args)
    return logits[:, 0], scale[:, 0]


@functools.partial(
    pl.kernel,
    out_type=jax.ShapeDtypeStruct((NC * NPAD,), _f32),
    mesh=_MESH,
    compiler_params=_SC_PARAMS,
    scratch_types=[
        pltpu.VMEM((NPAD,), _f32),
        pltpu.VMEM((ECHUNK,), _i32),
        pltpu.VMEM((ECHUNK,), _f32),
        pltpu.VMEM((16,), _i32),
        pltpu.VMEM((16,), _f32),
        pltpu.VMEM((NSLICE,), _f32),
        pltpu.VMEM((NSLICE,), _f32),
        pltpu.HBM((NC * NS * NPAD,), _f32),
    ],
)
def _sc_segmax(dst_hbm, logit_hbm, out_hbm,
               maxarr, dstv, lgv, kb, vb, row_v, acc_v, sp):
    cid = lax.axis_index("c")
    sid = lax.axis_index("s")
    base = _wid() * EP_TILE
    _fill(maxarr, NPAD, NEG, _f32)

    def chunk(i, _):
        off = base + i * ECHUNK
        pltpu.sync_copy(dst_hbm.at[pl.ds(off, ECHUNK)], dstv)
        pltpu.sync_copy(logit_hbm.at[pl.ds(off, ECHUNK)], lgv)

        def grp(g, _):
            s = pl.ds(g * 16, 16)
            _seg_rmw(maxarr, kb, vb, dstv[s], lgv[s], jnp.maximum)
            return 0

        lax.fori_loop(0, ECHUNK // 16, grp, 0)
        return 0

    lax.fori_loop(0, NCHUNK, chunk, 0)
    _merge_to_out(maxarr, sp, row_v, acc_v, out_hbm, cid, sid, jnp.maximum)


@functools.partial(
    pl.kernel,
    out_type=(
        jax.ShapeDtypeStruct((NC * NPAD,), _f32),
        jax.ShapeDtypeStruct((EPAD,), _f32),
    ),
    mesh=_MESH,
    compiler_params=_SC_PARAMS,
    scratch_types=[
        pltpu.VMEM((NPAD,), _f32),
        pltpu.VMEM((NPAD,), _f32),
        pltpu.VMEM((ECHUNK,), _i32),
        pltpu.VMEM((ECHUNK,), _f32),
        pltpu.VMEM((ECHUNK,), _f32),
        pltpu.VMEM((16,), _i32),
        pltpu.VMEM((16,), _f32),
        pltpu.VMEM((NSLICE,), _f32),
        pltpu.VMEM((NSLICE,), _f32),
        pltpu.HBM((NC * NS * NPAD,), _f32),
    ],
)
def _sc_segsum(dst_hbm, logit_hbm, maxpart_hbm, sum_hbm, ev_hbm,
               maxfull, sumarr, dstv, lgv, evv, kb, vb, row_v, acc_v, sp):
    cid = lax.axis_index("c")
    sid = lax.axis_index("s")
    base = _wid() * EP_TILE
    _merge_parts(maxpart_hbm, maxfull, row_v, jnp.maximum)
    _fill(sumarr, NPAD, 0.0, _f32)

    def chunk(i, _):
        off = base + i * ECHUNK
        pltpu.sync_copy(dst_hbm.at[pl.ds(off, ECHUNK)], dstv)
        pltpu.sync_copy(logit_hbm.at[pl.ds(off, ECHUNK)], lgv)

        def grp(g, _):
            s = pl.ds(g * 16, 16)
            k = dstv[s]
            m = plsc.load_gather(maxfull, [k])
            e = jnp.exp(lgv[s] - m)
            evv[s] = e
            _seg_rmw(sumarr, kb, vb, k, e, lambda a, b: a + b)
            return 0

        lax.fori_loop(0, ECHUNK // 16, grp, 0)
        pltpu.sync_copy(evv, ev_hbm.at[pl.ds(off, ECHUNK)])
        return 0

    lax.fori_loop(0, NCHUNK, chunk, 0)
    _merge_to_out(sumarr, sp, row_v, acc_v, sum_hbm, cid, sid,
                  lambda a, b: a + b)


@functools.partial(
    pl.kernel,
    out_type=jax.ShapeDtypeStruct((EPAD,), _f32),
    mesh=_MESH,
    compiler_params=_SC_PARAMS,
    scratch_types=[
        pltpu.VMEM((NPAD,), _f32),
        pltpu.VMEM((ECHUNK,), _i32),
        pltpu.VMEM((ECHUNK,), _f32),
        pltpu.VMEM((ECHUNK,), _f32),
        pltpu.VMEM((ECHUNK,), _f32),
        pltpu.VMEM((NSLICE,), _f32),
    ],
)
def _sc_weights(dst_hbm, ev_hbm, sc_hbm, sumpart_hbm, w_hbm,
                sumfull, dstv, evv, scv, wv, row_v):
    base = _wid() * EP_TILE
    _merge_parts(sumpart_hbm, sumfull, row_v, lambda a, b: a + b)

    def chunk(i, _):
        off = base + i * ECHUNK
        pltpu.sync_copy(dst_hbm.at[pl.ds(off, ECHUNK)], dstv)
        pltpu.sync_copy(ev_hbm.at[pl.ds(off, ECHUNK)], evv)
        pltpu.sync_copy(sc_hbm.at[pl.ds(off, ECHUNK)], scv)

        def grp(g, _):
            s = pl.ds(g * 16, 16)
            ssum = plsc.load_gather(sumfull, [dstv[s]])
            wv[s] = evv[s] / (ssum + 1e-8) * scv[s]
            return 0

        lax.fori_loop(0, ECHUNK // 16, grp, 0)
        pltpu.sync_copy(wv, w_hbm.at[pl.ds(off, ECHUNK)])
        return 0

    lax.fori_loop(0, NCHUNK, chunk, 0)


def _tc_rows_body(w, hx, out):
    out[...] = w[...] * hx[:, 0:H]


def _tc_rows(w, hx):
    return pl.pallas_call(
        _tc_rows_body,
        grid=(NB,),
        in_specs=[pl.BlockSpec((BE, 1), lambda i: (i, 0)),
                  pl.BlockSpec((BE, 128), lambda i: (i, 0))],
        out_specs=pl.BlockSpec((BE, H), lambda i: (i, 0)),
        out_shape=jax.ShapeDtypeStruct((EPAD, H), _f32),
    )(w[:, None], hx)


@functools.partial(
    pl.kernel,
    out_type=jax.ShapeDtypeStruct((NPAD, H), _f32),
    mesh=_MESH,
    compiler_params=_SC_PARAMS_UNTILED,
    scratch_types=[
        pltpu.VMEM((GCHUNK,), _i32),
        pltpu.VMEM((GCHUNK,), _i32),
        pltpu.VMEM((GCHUNK, H), _f32),
        pltpu.VMEM((GCHUNK, H), _f32),
        pltpu.VMEM_SHARED((PB_ROWS, H), _f32),
    ],
)
def _sc_scatter(dst_hbm, wrow_hbm, out_hbm, dstv, idxv, rows_v, zero_v, sp):
    cid = lax.axis_index("c")
    sid = lax.axis_index("s")
    _fill_rows(zero_v)
    rows_per_tile = PB_ROWS // NS
    n_full = rows_per_tile // GCHUNK
    rbase = sid * rows_per_tile
    for b in range(n_full):
        pltpu.sync_copy(zero_v, sp.at[pl.ds(rbase + b * GCHUNK, GCHUNK)])
    rem = rows_per_tile - n_full * GCHUNK
    if rem:
        pltpu.sync_copy(zero_v.at[pl.ds(0, rem)],
                        sp.at[pl.ds(rbase + n_full * GCHUNK, rem)])
    plsc.subcore_barrier()

    lo = cid * HALF
    tile_edges = EPAD // NS
    base = sid * tile_edges

    def chunk(i, _):
        off = base + i * GCHUNK
        pltpu.sync_copy(dst_hbm.at[pl.ds(off, GCHUNK)], dstv)
        pltpu.sync_copy(wrow_hbm.at[pl.ds(off, GCHUNK)], rows_v)

        def grp(g, _):
            s = pl.ds(g * 16, 16)
            li = dstv[s] - lo
            ok = (li >= 0) & (li < HALF)
            idxv[s] = jnp.where(ok, li, HALF + sid)
            return 0

        lax.fori_loop(0, GCHUNK // 16, grp, 0)
        pltpu.sync_copy(rows_v, sp.at[idxv], add=True)
        return 0

    lax.fori_loop(0, tile_edges // GCHUNK, chunk, 0)
    plsc.subcore_barrier()
    obase = cid * HALF + sid * ROWS_TILE
    pltpu.sync_copy(sp.at[pl.ds(sid * ROWS_TILE, ROWS_TILE)],
                    out_hbm.at[pl.ds(obase, ROWS_TILE)])


def _fill_rows(ref):
    zeros = jnp.zeros((16,), _f32)

    def body(i, _):
        r = i // 4
        c = (i % 4) * 16
        ref[r, pl.ds(c, 16)] = zeros
        return 0

    lax.fori_loop(0, GCHUNK * 4, body, 0)


def kernel(h, lice, node_features, edge_index, edge_distance, edge_direction,
           We1, be1, We2, be2, Wa1, ba1, Wa2, ba2, log_beta):
    src = edge_index[0]
    dst = edge_index[1]
    pad = EPAD - E
    srcp = jnp.pad(src, (0, pad))
    dstp = jnp.pad(dst, (0, pad))
    distp = jnp.pad(edge_distance, (0, pad))
    d0 = jnp.pad(edge_direction[:, 0], (0, pad))
    d1 = jnp.pad(edge_direction[:, 1], (0, pad))
    beta = jnp.exp(log_beta)
    htab = jnp.concatenate(
        [h, node_features[:, 11:15], lice[:, 0:1] * beta,
         jnp.zeros((N, 59), _f32)], axis=1)

    hx = _sc_gather(srcp, htab)
    logits, scale = _tc_logits(distp, d0, d1, hx, We1, be1, We2, be2,
                               Wa1, ba1, Wa2, ba2)
    maxpart = _sc_segmax(dstp, logits)
    sumpart, evals = _sc_segsum(dstp, logits, maxpart)
    w = _sc_weights(dstp, evals, scale, sumpart)
    weighted = _tc_rows(w, hx)
    pressure = _sc_scatter(dstp, weighted)
    return pressure[:N]


# speedup vs baseline: 9.9211x; 1.1400x over previous
import functools

import jax
import jax.numpy as jnp
from jax import lax
from jax.experimental import pallas as pl
from jax.experimental.pallas import tpu as pltpu
from jax.experimental.pallas import tpu_sc as plsc

N = 50000
E = 800000
H = 64
DECAY_KM = 15.0
NEG = -3.0e38

NC = 2
NS = 16
NW = NC * NS

EP_TILE = 25088
EPAD = EP_TILE * NW
ECHUNK = 512
NCHUNK = EP_TILE // ECHUNK
GCHUNK = 128

NPAD = 51200
NSLICE = NPAD // NS
NGRP = NSLICE // 16

BE = 2048
NB = EPAD // BE

HALF = NPAD // NC
ROWS_TILE = HALF // NS
PB_ROWS = HALF + 128

_MESH = plsc.VectorSubcoreMesh(core_axis_name="c", subcore_axis_name="s")
_SC_PARAMS = pltpu.CompilerParams(needs_layout_passes=False)
_SC_PARAMS_UNTILED = pltpu.CompilerParams(
    needs_layout_passes=False, use_tc_tiling_on_sc=False)

_f32 = jnp.float32
_i32 = jnp.int32


def _wid():
    return lax.axis_index("s") * NC + lax.axis_index("c")


def _lane():
    return lax.iota(_i32, 16)


def _fill(ref, n, value, dtype):
    v = jnp.full((16,), value, dtype)

    def body(i, _):
        ref[pl.ds(i * 16, 16)] = v
        return 0

    lax.fori_loop(0, n // 16, body, 0)


def _seg_rmw(arr_ref, kb_ref, vb_ref, k, v, op):
    ks, vs = plsc.sort_key_val(k, v)
    lane = _lane()
    kb_ref[...] = ks
    for off in (1, 2, 4, 8):
        idx = jnp.maximum(lane - off, 0)
        vb_ref[...] = vs
        ksh = plsc.load_gather(kb_ref, [idx])
        vsh = plsc.load_gather(vb_ref, [idx])
        vs = jnp.where((ksh == ks) & (lane >= off), op(vs, vsh), vs)
    knext = plsc.load_gather(kb_ref, [jnp.minimum(lane + 1, 15)])
    is_last = (knext != ks) | (lane == 15)
    old = plsc.load_gather(arr_ref, [ks])
    plsc.store_scatter(arr_ref, [ks], op(old, vs), mask=is_last)


def _merge_to_out(arr_ref, sp_ref, row_ref, acc_ref, out_ref, cid, sid, op):
    stage = cid * NS * NPAD
    pltpu.sync_copy(arr_ref, sp_ref.at[pl.ds(stage + sid * NPAD, NPAD)])
    plsc.subcore_barrier()
    base = sid * NSLICE
    pltpu.sync_copy(sp_ref.at[pl.ds(stage + base, NSLICE)], acc_ref)
    for r in range(1, NS):
        pltpu.sync_copy(sp_ref.at[pl.ds(stage + r * NPAD + base, NSLICE)],
                        row_ref)

        def body(j, _):
            s = pl.ds(j * 16, 16)
            acc_ref[s] = op(acc_ref[s], row_ref[s])
            return 0

        lax.fori_loop(0, NGRP, body, 0)
    pltpu.sync_copy(acc_ref, out_ref.at[pl.ds(cid * NPAD + base, NSLICE)])


def _merge_parts(part_hbm, full_ref, row_ref, op):
    pltpu.sync_copy(part_hbm.at[pl.ds(0, NPAD)], full_ref)
    for r in range(NS):
        pltpu.sync_copy(part_hbm.at[pl.ds(NPAD + r * NSLICE, NSLICE)], row_ref)

        def body(j, _):
            s = pl.ds(r * NSLICE + j * 16, 16)
            t = pl.ds(j * 16, 16)
            full_ref[s] = op(full_ref[s], row_ref[t])
            return 0

        lax.fori_loop(0, NGRP, body, 0)


@functools.partial(
    pl.kernel,
    out_type=jax.ShapeDtypeStruct((EPAD, 128), _f32),
    mesh=_MESH,
    compiler_params=_SC_PARAMS,
    scratch_types=[
        pltpu.VMEM((GCHUNK,), _i32),
        pltpu.VMEM((GCHUNK,), _i32),
        pltpu.VMEM((GCHUNK, 128), _f32),
        pltpu.VMEM((GCHUNK, 128), _f32),
        pltpu.SemaphoreType.DMA,
        pltpu.SemaphoreType.DMA,
    ],
)
def _sc_gather(src_hbm, htab_hbm, hx_hbm, idx0, idx1, rows0, rows1,
               sem0, sem1):
    base = _wid() * EP_TILE
    niter = EP_TILE // GCHUNK
    idx = (idx0, idx1)
    rows = (rows0, rows1)
    sems = (sem0, sem1)
    last = base + (niter - 1) * GCHUNK

    pltpu.sync_copy(src_hbm.at[pl.ds(base, GCHUNK)], idx0)
    pltpu.async_copy(htab_hbm.at[idx0], rows0, sem0)
    pltpu.sync_copy(src_hbm.at[pl.ds(base + GCHUNK, GCHUNK)], idx1)

    def pair(p, _):
        for b in (0, 1):
            j = p * 2 + b
            off = base + j * GCHUNK
            pltpu.make_async_copy(htab_hbm.at[idx[b]], rows[b],
                                  sems[b]).wait()
            pltpu.async_copy(htab_hbm.at[idx[1 - b]], rows[1 - b],
                             sems[1 - b])
            pltpu.sync_copy(src_hbm.at[pl.ds(jnp.minimum(
                off + 2 * GCHUNK, last), GCHUNK)], idx[b])
            pltpu.sync_copy(rows[b], hx_hbm.at[pl.ds(off, GCHUNK)])
        return 0

    lax.fori_loop(0, niter // 2, pair, 0)
    pltpu.make_async_copy(htab_hbm.at[idx0], rows0, sems[0]).wait()


def _tc_logits_body(dist, d0, d1, hx, we1, be1, we2, be2,
                    wa1e, wa1h, ba1, wa2, ba2, out, scale_out):
    temp = hx[:, 64:65]
    sal = hx[:, 65:66]
    u = hx[:, 66:67]
    v = hx[:, 67:68]
    hsrc = hx[:, 0:H]
    flux = u * d0[...] + v * d1[...]
    wf = jnp.maximum(flux, 0.0) * jnp.exp(-dist[...] / DECAY_KM)
    e1 = jnp.maximum(
        dist[...] * we1[0:1, :] + wf * we1[1:2, :]
        + temp * we1[2:3, :] + sal * we1[3:4, :] + be1[...], 0.0)
    w_ea = jnp.dot(we2[...], wa1e[...], preferred_element_type=_f32)
    b_fold = jnp.dot(be2[...], wa1e[...], preferred_element_type=_f32) + ba1[...]
    att = jnp.maximum(
        jnp.dot(e1, w_ea, preferred_element_type=_f32)
        + jnp.dot(hsrc, wa1h[...], preferred_element_type=_f32)
        + b_fold, 0.0)
    logit = (jnp.sum(att * wa2[...], axis=1, keepdims=True) + ba2[...]
             + jnp.log(wf + 1e-8))
    rid = pl.program_id(0) * BE + lax.broadcasted_iota(_i32, (BE, 1), 0)
    out[...] = jnp.where(rid < E, logit, NEG)
    scale_out[...] = hx[:, 68:69]


def _tc_logits(dist, d0, d1, hx, We1, be1, We2, be2, Wa1, ba1, Wa2, ba2):
    col = pl.BlockSpec((BE, 1), lambda i: (i, 0))
    full = lambda a: pl.BlockSpec(a.shape, lambda i: tuple(0 for _ in a.shape))
    args = (dist[:, None], d0[:, None], d1[:, None], hx,
            We1, be1[None, :], We2, be2[None, :],
            Wa1[:H], Wa1[H:], ba1[None, :], Wa2[:, 0][None, :],
            ba2[None, :])
    logits, scale = pl.pallas_call(
        _tc_logits_body,
        grid=(NB,),
        in_specs=[col, col, col,
                  pl.BlockSpec((BE, 128), lambda i: (i, 0)),
                  full(args[4]), full(args[5]), full(args[6]), full(args[7]),
                  full(args[8]), full(args[9]), full(args[10]),
                  full(args[11]), full(args[12])],
        out_specs=(col, col),
        out_shape=(jax.ShapeDtypeStruct((EPAD, 1), _f32),
                   jax.ShapeDtypeStruct((EPAD, 1), _f32)),
    )(*args)
    return logits[:, 0], scale[:, 0]


@functools.partial(
    pl.kernel,
    out_type=jax.ShapeDtypeStruct((NC * NPAD,), _f32),
    mesh=_MESH,
    compiler_params=_SC_PARAMS,
    scratch_types=[
        pltpu.VMEM((NPAD,), _f32),
        pltpu.VMEM((ECHUNK,), _i32),
        pltpu.VMEM((ECHUNK,), _f32),
        pltpu.VMEM((16,), _i32),
        pltpu.VMEM((16,), _f32),
        pltpu.VMEM((NSLICE,), _f32),
        pltpu.VMEM((NSLICE,), _f32),
        pltpu.HBM((NC * NS * NPAD,), _f32),
    ],
)
def _sc_segmax(dst_hbm, logit_hbm, out_hbm,
               maxarr, dstv, lgv, kb, vb, row_v, acc_v, sp):
    cid = lax.axis_index("c")
    sid = lax.axis_index("s")
    base = _wid() * EP_TILE
    _fill(maxarr, NPAD, NEG, _f32)

    def chunk(i, _):
        off = base + i * ECHUNK
        pltpu.sync_copy(dst_hbm.at[pl.ds(off, ECHUNK)], dstv)
        pltpu.sync_copy(logit_hbm.at[pl.ds(off, ECHUNK)], lgv)

        def grp(g, _):
            s = pl.ds(g * 16, 16)
            _seg_rmw(maxarr, kb, vb, dstv[s], lgv[s], jnp.maximum)
            return 0

        lax.fori_loop(0, ECHUNK // 16, grp, 0)
        return 0

    lax.fori_loop(0, NCHUNK, chunk, 0)
    _merge_to_out(maxarr, sp, row_v, acc_v, out_hbm, cid, sid, jnp.maximum)


@functools.partial(
    pl.kernel,
    out_type=(
        jax.ShapeDtypeStruct((NC * NPAD,), _f32),
        jax.ShapeDtypeStruct((EPAD,), _f32),
    ),
    mesh=_MESH,
    compiler_params=_SC_PARAMS,
    scratch_types=[
        pltpu.VMEM((NPAD,), _f32),
        pltpu.VMEM((NPAD,), _f32),
        pltpu.VMEM((ECHUNK,), _i32),
        pltpu.VMEM((ECHUNK,), _f32),
        pltpu.VMEM((ECHUNK,), _f32),
        pltpu.VMEM((ECHUNK,), _f32),
        pltpu.VMEM((16,), _i32),
        pltpu.VMEM((16,), _f32),
        pltpu.VMEM((NSLICE,), _f32),
        pltpu.VMEM((NSLICE,), _f32),
        pltpu.HBM((NC * NS * NPAD,), _f32),
    ],
)
def _sc_segsum(dst_hbm, logit_hbm, sc_hbm, maxpart_hbm, sum_hbm, ev_hbm,
               maxfull, sumarr, dstv, lgv, scv, evv, kb, vb, row_v, acc_v, sp):
    cid = lax.axis_index("c")
    sid = lax.axis_index("s")
    base = _wid() * EP_TILE
    _merge_parts(maxpart_hbm, maxfull, row_v, jnp.maximum)
    _fill(sumarr, NPAD, 0.0, _f32)

    def chunk(i, _):
        off = base + i * ECHUNK
        pltpu.sync_copy(dst_hbm.at[pl.ds(off, ECHUNK)], dstv)
        pltpu.sync_copy(logit_hbm.at[pl.ds(off, ECHUNK)], lgv)
        pltpu.sync_copy(sc_hbm.at[pl.ds(off, ECHUNK)], scv)

        def grp(g, _):
            s = pl.ds(g * 16, 16)
            k = dstv[s]
            m = plsc.load_gather(maxfull, [k])
            e = jnp.exp(lgv[s] - m)
            evv[s] = e * scv[s]
            _seg_rmw(sumarr, kb, vb, k, e, lambda a, b: a + b)
            return 0

        lax.fori_loop(0, ECHUNK // 16, grp, 0)
        pltpu.sync_copy(evv, ev_hbm.at[pl.ds(off, ECHUNK)])
        return 0

    lax.fori_loop(0, NCHUNK, chunk, 0)
    _merge_to_out(sumarr, sp, row_v, acc_v, sum_hbm, cid, sid,
                  lambda a, b: a + b)


def _tc_rows_body(w, hx, out):
    rid = pl.program_id(0) * BE + lax.broadcasted_iota(_i32, (BE, 1), 0)
    out[...] = jnp.where(rid < E, w[...], 0.0) * hx[:, 0:H]


def _tc_rows(w, hx):
    return pl.pallas_call(
        _tc_rows_body,
        grid=(NB,),
        in_specs=[pl.BlockSpec((BE, 1), lambda i: (i, 0)),
                  pl.BlockSpec((BE, 128), lambda i: (i, 0))],
        out_specs=pl.BlockSpec((BE, H), lambda i: (i, 0)),
        out_shape=jax.ShapeDtypeStruct((EPAD, H), _f32),
    )(w[:, None], hx)


HHALF = H // NC

@functools.partial(
    pl.kernel,
    out_type=jax.ShapeDtypeStruct((NPAD, H), _f32),
    mesh=_MESH,
    compiler_params=_SC_PARAMS_UNTILED,
    scratch_types=[
        pltpu.VMEM((GCHUNK,), _i32),
        pltpu.VMEM((GCHUNK,), _i32),
        pltpu.VMEM((GCHUNK, HHALF), _f32),
        pltpu.VMEM((GCHUNK, HHALF), _f32),
        pltpu.VMEM((GCHUNK,), _f32),
        pltpu.VMEM((GCHUNK,), _f32),
        pltpu.SemaphoreType.DMA,
        pltpu.SemaphoreType.DMA,
        pltpu.VMEM_SHARED((NPAD, HHALF), _f32),
    ],
)
def _sc_scatter(dst_hbm, wrow_hbm, sumpart_hbm, out_hbm,
                dst0, dst1, rows0, rows1, s0_v, s1_v, sem0, sem1, sp):
    cid = lax.axis_index("c")
    sid = lax.axis_index("s")
    cbase = cid * HHALF
    _fill_rows(rows0)
    for zb in range(NSLICE // GCHUNK):
        pltpu.sync_copy(
            rows0, sp.at[pl.ds(sid * NSLICE + zb * GCHUNK, GCHUNK)])
    plsc.subcore_barrier()

    tile_edges = EPAD // NS
    niter = tile_edges // GCHUNK
    base = sid * tile_edges
    dstb = (dst0, dst1)
    rowsb = (rows0, rows1)
    sems = (sem0, sem1)

    pltpu.sync_copy(dst_hbm.at[pl.ds(base, GCHUNK)], dst0)
    pltpu.async_copy(
        wrow_hbm.at[pl.ds(base, GCHUNK), pl.ds(cbase, HHALF)], rows0, sem0)

    def pair(p, _):
        for b in (0, 1):
            j = p * 2 + b
            off = base + j * GCHUNK
            nxt = jnp.minimum(off + GCHUNK, base + (niter - 1) * GCHUNK)
            pltpu.sync_copy(dst_hbm.at[pl.ds(nxt, GCHUNK)], dstb[1 - b])
            pltpu.async_copy(
                wrow_hbm.at[pl.ds(nxt, GCHUNK), pl.ds(cbase, HHALF)],
                rowsb[1 - b], sems[1 - b])
            pltpu.make_async_copy(
                wrow_hbm.at[pl.ds(off, GCHUNK), pl.ds(cbase, HHALF)],
                rowsb[b], sems[b]).wait()
            pltpu.sync_copy(rowsb[b], sp.at[dstb[b]], add=True)
        return 0

    lax.fori_loop(0, niter // 2, pair, 0)
    pltpu.make_async_copy(
        wrow_hbm.at[pl.ds(base, GCHUNK), pl.ds(cbase, HHALF)],
        rows0, sems[0]).wait()
    plsc.subcore_barrier()

    for nb in range(NSLICE // GCHUNK):
        nbase = sid * NSLICE + nb * GCHUNK
        pltpu.sync_copy(sumpart_hbm.at[pl.ds(nbase, GCHUNK)], s0_v)
        pltpu.sync_copy(sumpart_hbm.at[pl.ds(NPAD + nbase, GCHUNK)], s1_v)
        pltpu.sync_copy(sp.at[pl.ds(nbase, GCHUNK)], rows0)

        def inv_body(i, _):
            t = pl.ds(i * 16, 16)
            s0_v[t] = 1.0 / (s0_v[t] + s1_v[t] + 1e-8)
            return 0

        lax.fori_loop(0, GCHUNK // 16, inv_body, 0)

        def norm_grp(g, _):
            inv16 = s0_v[pl.ds(g * 16, 16)]
            for rr in range(16):
                f = inv16[rr]
                r = g * 16 + rr
                for cc in range(HHALF // 16):
                    t = pl.ds(cc * 16, 16)
                    rows0[r, t] = rows0[r, t] * f
            return 0

        lax.fori_loop(0, GCHUNK // 16, norm_grp, 0)
        pltpu.sync_copy(
            rows0, out_hbm.at[pl.ds(nbase, GCHUNK), pl.ds(cbase, HHALF)])


def _fill_rows(ref):
    zeros = jnp.zeros((16,), _f32)
    nrow, ncol = ref.shape
    per_row = ncol // 16

    def body(i, _):
        r = i // per_row
        c = (i % per_row) * 16
        ref[r, pl.ds(c, 16)] = zeros
        return 0

    lax.fori_loop(0, nrow * per_row, body, 0)


def kernel(h, lice, node_features, edge_index, edge_distance, edge_direction,
           We1, be1, We2, be2, Wa1, ba1, Wa2, ba2, log_beta):
    src = edge_index[0]
    dst = edge_index[1]
    pad = EPAD - E
    srcp = jnp.pad(src, (0, pad))
    dstp = jnp.pad(dst, (0, pad))
    distp = jnp.pad(edge_distance, (0, pad))
    d0 = jnp.pad(edge_direction[:, 0], (0, pad))
    d1 = jnp.pad(edge_direction[:, 1], (0, pad))
    beta = jnp.exp(log_beta)
    htab = jnp.concatenate(
        [h, node_features[:, 11:15], lice[:, 0:1] * beta,
         jnp.zeros((N, 59), _f32)], axis=1)

    hx = _sc_gather(srcp, htab)
    logits, scale = _tc_logits(distp, d0, d1, hx, We1, be1, We2, be2,
                               Wa1, ba1, Wa2, ba2)
    maxpart = _sc_segmax(dstp, logits)
    sumpart, ev2 = _sc_segsum(dstp, logits, scale, maxpart)
    weighted = _tc_rows(ev2, hx)
    pressure = _sc_scatter(dstp, weighted, sumpart)
    return pressure[:N]
